# initial kernel scaffold (unmeasured)
import jax
import jax.numpy as jnp
from jax import lax
from jax.experimental import pallas as pl
from jax.experimental.pallas import tpu as pltpu

N_DEV = 4
SQ = 2048
DM = 1024
HQ = 32
HG = 8
DH = 128
SCALE = 0.08838834764831843
NHOP = N_DEV - 1


def _attn_head(q, k, v):
    q4 = q.reshape(8, 4, 64, DH)
    k4 = k.reshape(8, 4, 64, DH)
    v4 = v.reshape(8, 4, 64, DH)
    parts = []
    for r in range(4):
        qr = q4[:, r].reshape(512, DH)
        kr = k4[:, r].reshape(512, DH)
        vr = v4[:, r].reshape(512, DH)
        s = lax.dot_general(
            qr, kr, (((1,), (1,)), ((), ())),
            preferred_element_type=jnp.float32,
        ) * SCALE
        m = jnp.max(s, axis=1, keepdims=True)
        p = jnp.exp(s - m)
        den = jnp.sum(p, axis=1, keepdims=True)
        c = jnp.dot(p, vr, preferred_element_type=jnp.float32) / den
        parts.append(c.reshape(8, 64, DH))
    return jnp.stack(parts, axis=1).reshape(SQ, DH)


def kernel(x, Wq, K_ext, V_ext, Wo):
    x2 = x.reshape(SQ, DM)

    def body(x_ref, wq_ref, k_ref, v_ref, wo_ref, out_ref,
             gq, go, kbuf, vbuf, init_sems, kv_sems, send_sems, recv_sems):
        my = lax.axis_index("i")
        left = lax.rem(my + N_DEV - 1, N_DEV)
        right = lax.rem(my + 1, N_DEV)

        for h in range(HG):
            pltpu.make_async_copy(
                wq_ref.at[:, pl.ds(h * DH, DH)], gq.at[my, h], init_sems.at[0, h]
            ).start()
            pltpu.make_async_copy(
                wo_ref.at[pl.ds(h * DH, DH), :], go.at[my, h], init_sems.at[1, h]
            ).start()
        for h in range(HG):
            pltpu.make_async_copy(
                wq_ref.at[:, pl.ds(h * DH, DH)], gq.at[my, h], init_sems.at[0, h]
            ).wait()
            pltpu.make_async_copy(
                wo_ref.at[pl.ds(h * DH, DH), :], go.at[my, h], init_sems.at[1, h]
            ).wait()

        barrier_sem = pltpu.get_barrier_semaphore()
        for nbr in (left, right):
            pl.semaphore_signal(
                barrier_sem, inc=1,
                device_id=(nbr,), device_id_type=pl.DeviceIdType.MESH,
            )
        pl.semaphore_wait(barrier_sem, 2)

        for t in range(NHOP):
            org = lax.rem(my - t + N_DEV, N_DEV)
            rq = pltpu.make_async_remote_copy(
                src_ref=gq.at[org], dst_ref=gq.at[org],
                send_sem=send_sems.at[t, 0], recv_sem=recv_sems.at[t, 0],
                device_id=(right,), device_id_type=pl.DeviceIdType.MESH,
            )
            ro = pltpu.make_async_remote_copy(
                src_ref=go.at[org], dst_ref=go.at[org],
                send_sem=send_sems.at[t, 1], recv_sem=recv_sems.at[t, 1],
                device_id=(right,), device_id_type=pl.DeviceIdType.MESH,
            )
            rq.start()
            ro.start()
            rq.wait()
            ro.wait()

        out_ref[...] = jnp.zeros((SQ, DM), jnp.float32)

        def head_body(g, carry):
            j = lax.div(g, HG)
            h = lax.rem(g, HG)
            ck = pltpu.make_async_copy(k_ref.at[my, :, g, :], kbuf, kv_sems.at[0])
            cv = pltpu.make_async_copy(v_ref.at[my, :, g, :], vbuf, kv_sems.at[1])
            ck.start()
            cv.start()
            q = jnp.dot(x_ref[...], gq[j, h], preferred_element_type=jnp.float32)
            ck.wait()
            cv.wait()
            ctx = _attn_head(q, kbuf[...], vbuf[...])
            out_ref[...] += jnp.dot(ctx, go[j, h], preferred_element_type=jnp.float32)
            return carry

        lax.fori_loop(0, HQ, head_body, 0)

    out = pl.pallas_call(
        body,
        out_shape=jax.ShapeDtypeStruct((SQ, DM), jnp.float32),
        in_specs=[
            pl.BlockSpec(memory_space=pltpu.VMEM),
            pl.BlockSpec(memory_space=pltpu.ANY),
            pl.BlockSpec(memory_space=pltpu.ANY),
            pl.BlockSpec(memory_space=pltpu.ANY),
            pl.BlockSpec(memory_space=pltpu.ANY),
        ],
        out_specs=pl.BlockSpec(memory_space=pltpu.VMEM),
        scratch_shapes=[
            pltpu.VMEM((N_DEV, HG, DM, DH), jnp.float32),
            pltpu.VMEM((N_DEV, HG, DH, DM), jnp.float32),
            pltpu.VMEM((SQ, DH), jnp.float32),
            pltpu.VMEM((SQ, DH), jnp.float32),
            pltpu.SemaphoreType.DMA((2, HG)),
            pltpu.SemaphoreType.DMA((2,)),
            pltpu.SemaphoreType.DMA((NHOP, 2)),
            pltpu.SemaphoreType.DMA((NHOP, 2)),
        ],
        compiler_params=pltpu.CompilerParams(collective_id=0),
    )(x2, Wq, K_ext, V_ext, Wo)
    return out.reshape(1, SQ, DM)


# baseline (device time: 542261 ns/iter reference)
import jax
import jax.numpy as jnp
from jax import lax
from jax.experimental import pallas as pl
from jax.experimental.pallas import tpu as pltpu

N_DEV = 4
SQ = 2048
DM = 1024
HQ = 32
HG = 8
DH = 128
SCALE = 0.08838834764831843
NHOP = N_DEV - 1


def _attn_head(q, k, v):
    q4 = q.reshape(8, 4, 64, DH)
    k4 = k.reshape(8, 4, 64, DH)
    v4 = v.reshape(8, 4, 64, DH)
    parts = []
    for r in range(4):
        qr = q4[:, r].reshape(512, DH)
        kr = k4[:, r].reshape(512, DH)
        vr = v4[:, r].reshape(512, DH)
        s = lax.dot_general(
            qr, kr, (((1,), (1,)), ((), ())),
            preferred_element_type=jnp.float32,
        ) * SCALE
        m = jnp.max(s, axis=1, keepdims=True)
        p = jnp.exp(s - m)
        den = jnp.sum(p, axis=1, keepdims=True)
        c = jnp.dot(p, vr, preferred_element_type=jnp.float32) / den
        parts.append(c.reshape(8, 64, DH))
    return jnp.stack(parts, axis=1).reshape(SQ, DH)


def kernel(x, Wq, K_ext, V_ext, Wo):
    x2 = x.reshape(SQ, DM)

    def body(x_ref, wq_ref, k_ref, v_ref, wo_ref, out_ref,
             gq, go, kbuf, vbuf, init_sems, kv_sems, send_sems, recv_sems):
        my = lax.axis_index("i")
        left = lax.rem(my + N_DEV - 1, N_DEV)
        right = lax.rem(my + 1, N_DEV)

        for h in range(HG):
            pltpu.make_async_copy(
                wq_ref.at[:, pl.ds(h * DH, DH)], gq.at[my, h], init_sems.at[0, h]
            ).start()
            pltpu.make_async_copy(
                wo_ref.at[pl.ds(h * DH, DH), :], go.at[my, h], init_sems.at[1, h]
            ).start()
        for h in range(HG):
            pltpu.make_async_copy(
                wq_ref.at[:, pl.ds(h * DH, DH)], gq.at[my, h], init_sems.at[0, h]
            ).wait()
            pltpu.make_async_copy(
                wo_ref.at[pl.ds(h * DH, DH), :], go.at[my, h], init_sems.at[1, h]
            ).wait()

        barrier_sem = pltpu.get_barrier_semaphore()
        for nbr in (left, right):
            pl.semaphore_signal(
                barrier_sem, inc=1,
                device_id=(nbr,), device_id_type=pl.DeviceIdType.MESH,
            )
        pl.semaphore_wait(barrier_sem, 2)

        for t in range(NHOP):
            org = lax.rem(my - t + N_DEV, N_DEV)
            rq = pltpu.make_async_remote_copy(
                src_ref=gq.at[org], dst_ref=gq.at[org],
                send_sem=send_sems.at[t, 0], recv_sem=recv_sems.at[t, 0],
                device_id=(right,), device_id_type=pl.DeviceIdType.MESH,
            )
            ro = pltpu.make_async_remote_copy(
                src_ref=go.at[org], dst_ref=go.at[org],
                send_sem=send_sems.at[t, 1], recv_sem=recv_sems.at[t, 1],
                device_id=(right,), device_id_type=pl.DeviceIdType.MESH,
            )
            rq.start()
            ro.start()
            rq.wait()
            ro.wait()

        out_ref[...] = jnp.zeros((SQ, DM), jnp.float32)

        def head_body(g, carry):
            j = lax.div(g, HG)
            h = lax.rem(g, HG)
            ck = pltpu.make_async_copy(k_ref.at[my, :, g, :], kbuf, kv_sems.at[0])
            cv = pltpu.make_async_copy(v_ref.at[my, :, g, :], vbuf, kv_sems.at[1])
            ck.start()
            cv.start()
            q = jnp.dot(x_ref[...], gq[j, h], preferred_element_type=jnp.float32)
            ck.wait()
            cv.wait()
            ctx = _attn_head(q, kbuf[...], vbuf[...])
            out_ref[...] += jnp.dot(ctx, go[j, h], preferred_element_type=jnp.float32)
            return carry

        lax.fori_loop(0, HQ, head_body, 0)

    out = pl.pallas_call(
        body,
        out_shape=jax.ShapeDtypeStruct((SQ, DM), jnp.float32),
        in_specs=[
            pl.BlockSpec(memory_space=pltpu.MemorySpace.VMEM),
            pl.BlockSpec(memory_space=pl.ANY),
            pl.BlockSpec(memory_space=pl.ANY),
            pl.BlockSpec(memory_space=pl.ANY),
            pl.BlockSpec(memory_space=pl.ANY),
        ],
        out_specs=pl.BlockSpec(memory_space=pltpu.MemorySpace.VMEM),
        scratch_shapes=[
            pltpu.VMEM((N_DEV, HG, DM, DH), jnp.float32),
            pltpu.VMEM((N_DEV, HG, DH, DM), jnp.float32),
            pltpu.VMEM((SQ, DH), jnp.float32),
            pltpu.VMEM((SQ, DH), jnp.float32),
            pltpu.SemaphoreType.DMA((2, HG)),
            pltpu.SemaphoreType.DMA((2,)),
            pltpu.SemaphoreType.DMA((NHOP, 2)),
            pltpu.SemaphoreType.DMA((NHOP, 2)),
        ],
        compiler_params=pltpu.CompilerParams(
            collective_id=0,
            vmem_limit_bytes=60 * 1024 * 1024,
        ),
    )(x2, Wq, K_ext, V_ext, Wo)
    return out.reshape(1, SQ, DM)


# device time: 216412 ns/iter; 2.5057x vs baseline; 2.5057x over previous
import jax
import jax.numpy as jnp
from jax import lax
from jax.experimental import pallas as pl
from jax.experimental.pallas import tpu as pltpu

N_DEV = 4
SQ = 2048
DM = 1024
HQ = 32
HG = 8
HH = HG // 2
DH = 128
SCALE = 0.08838834764831843
NHOP = N_DEV - 1


def _attn_head(q, k, v):
    q4 = q.reshape(8, 4, 64, DH)
    k4 = k.reshape(8, 4, 64, DH)
    v4 = v.reshape(8, 4, 64, DH)
    parts = []
    for r in range(4):
        qr = q4[:, r].reshape(512, DH)
        kr = k4[:, r].reshape(512, DH)
        vr = v4[:, r].reshape(512, DH)
        s = lax.dot_general(
            qr, kr, (((1,), (1,)), ((), ())),
            preferred_element_type=jnp.float32,
        ) * SCALE
        m = jnp.max(s, axis=1, keepdims=True)
        p = jnp.exp(s - m)
        den = jnp.sum(p, axis=1, keepdims=True)
        c = jnp.dot(p, vr, preferred_element_type=jnp.float32) / den
        parts.append(c.reshape(8, 64, DH))
    return jnp.stack(parts, axis=1).reshape(SQ, DH)


def kernel(x, Wq, K_ext, V_ext, Wo):
    x2 = x.reshape(SQ, DM)

    def body(x_ref, wq_ref, k_ref, v_ref, wo_ref, out_ref,
             gq, go, kbuf, vbuf, init_sems, kv_sems, send_sems, recv_sems):
        my = lax.axis_index("i")
        left = lax.rem(my + N_DEV - 1, N_DEV)
        right = lax.rem(my + 1, N_DEV)

        for h in range(HG):
            pltpu.make_async_copy(
                wq_ref.at[:, pl.ds(h * DH, DH)], gq.at[my, h], init_sems.at[0, h]
            ).start()
            pltpu.make_async_copy(
                wo_ref.at[pl.ds(h * DH, DH), :], go.at[my, h], init_sems.at[1, h]
            ).start()
        out_ref[...] = jnp.zeros((SQ, DM), jnp.float32)
        for h in range(HG):
            pltpu.make_async_copy(
                wq_ref.at[:, pl.ds(h * DH, DH)], gq.at[my, h], init_sems.at[0, h]
            ).wait()
            pltpu.make_async_copy(
                wo_ref.at[pl.ds(h * DH, DH), :], go.at[my, h], init_sems.at[1, h]
            ).wait()

        barrier_sem = pltpu.get_barrier_semaphore()
        for nbr in (left, right):
            pl.semaphore_signal(
                barrier_sem, inc=1,
                device_id=(nbr,), device_id_type=pl.DeviceIdType.MESH,
            )
        pl.semaphore_wait(barrier_sem, 2)

        def hop_rdmas(t):
            org_cw = lax.rem(my - t + N_DEV, N_DEV)
            org_ccw = lax.rem(my + t, N_DEV)
            rs = []
            for ti, ref in enumerate((gq, go)):
                rs.append(pltpu.make_async_remote_copy(
                    src_ref=ref.at[org_cw, pl.ds(0, HH)],
                    dst_ref=ref.at[org_cw, pl.ds(0, HH)],
                    send_sem=send_sems.at[t, 0, ti],
                    recv_sem=recv_sems.at[t, 0, ti],
                    device_id=(right,), device_id_type=pl.DeviceIdType.MESH,
                ))
                rs.append(pltpu.make_async_remote_copy(
                    src_ref=ref.at[org_ccw, pl.ds(HH, HH)],
                    dst_ref=ref.at[org_ccw, pl.ds(HH, HH)],
                    send_sem=send_sems.at[t, 1, ti],
                    recv_sem=recv_sems.at[t, 1, ti],
                    device_id=(left,), device_id_type=pl.DeviceIdType.MESH,
                ))
            return rs

        def compute_phase(p):
            def gh(h_i):
                if p == 0:
                    return my, h_i
                grp = jnp.where(
                    h_i < HH,
                    lax.rem(my - p + N_DEV, N_DEV),
                    lax.rem(my + p, N_DEV),
                )
                return grp, h_i

            def kv_copies(h_i, slot):
                grp, head = gh(h_i)
                g = grp * HG + head
                return (
                    pltpu.make_async_copy(
                        k_ref.at[my, :, g, :], kbuf.at[slot], kv_sems.at[slot, 0]
                    ),
                    pltpu.make_async_copy(
                        v_ref.at[my, :, g, :], vbuf.at[slot], kv_sems.at[slot, 1]
                    ),
                )

            for c in kv_copies(jnp.int32(0), 0):
                c.start()

            def head_body(h_i, carry):
                slot = lax.rem(h_i, 2)
                grp, head = gh(h_i)
                for c in kv_copies(h_i, slot):
                    c.wait()

                @pl.when(h_i < HG - 1)
                def _():
                    for c in kv_copies(h_i + 1, lax.rem(h_i + 1, 2)):
                        c.start()

                q = jnp.dot(x_ref[...], gq[grp, head],
                            preferred_element_type=jnp.float32)
                ctx = _attn_head(q, kbuf[slot], vbuf[slot])
                wo_h = go[grp, head]
                for lo in (0, SQ // 2):
                    out_ref[pl.ds(lo, SQ // 2), :] += jnp.dot(
                        ctx[lo:lo + SQ // 2], wo_h,
                        preferred_element_type=jnp.float32,
                    )
                return carry

            lax.fori_loop(0, HG, head_body, 0)

        for t in range(NHOP):
            for r in hop_rdmas(t):
                r.start()
            compute_phase(t)
            for r in hop_rdmas(t):
                r.wait()
        compute_phase(NHOP)

    out = pl.pallas_call(
        body,
        out_shape=jax.ShapeDtypeStruct((SQ, DM), jnp.float32),
        in_specs=[
            pl.BlockSpec(memory_space=pltpu.MemorySpace.VMEM),
            pl.BlockSpec(memory_space=pl.ANY),
            pl.BlockSpec(memory_space=pl.ANY),
            pl.BlockSpec(memory_space=pl.ANY),
            pl.BlockSpec(memory_space=pl.ANY),
        ],
        out_specs=pl.BlockSpec(memory_space=pltpu.MemorySpace.VMEM),
        scratch_shapes=[
            pltpu.VMEM((N_DEV, HG, DM, DH), jnp.float32),
            pltpu.VMEM((N_DEV, HG, DH, DM), jnp.float32),
            pltpu.VMEM((2, SQ, DH), jnp.float32),
            pltpu.VMEM((2, SQ, DH), jnp.float32),
            pltpu.SemaphoreType.DMA((2, HG)),
            pltpu.SemaphoreType.DMA((2, 2)),
            pltpu.SemaphoreType.DMA((NHOP, 2, 2)),
            pltpu.SemaphoreType.DMA((NHOP, 2, 2)),
        ],
        compiler_params=pltpu.CompilerParams(
            collective_id=0,
            vmem_limit_bytes=100 * 1024 * 1024,
        ),
    )(x2, Wq, K_ext, V_ext, Wo)
    return out.reshape(1, SQ, DM)


# device time: 127702 ns/iter; 4.2463x vs baseline; 1.6947x over previous
import jax
import jax.numpy as jnp
from jax import lax
from jax.experimental import pallas as pl
from jax.experimental.pallas import tpu as pltpu

N_DEV = 4
SQ = 2048
DM = 1024
HQ = 32
HG = 8
HH = HG // 2
HD = HH * 128
DH = 128
SCALE = 0.08838834764831843
NHOP = N_DEV - 1


def _attn_head(q, k, v):
    q4 = q.reshape(8, 4, 64, DH)
    k4 = k.reshape(8, 4, 64, DH)
    v4 = v.reshape(8, 4, 64, DH)
    parts = []
    for r in range(4):
        qr = q4[:, r].reshape(512, DH)
        kr = k4[:, r].reshape(512, DH)
        vr = v4[:, r].reshape(512, DH)
        s = lax.dot_general(
            qr, kr, (((1,), (1,)), ((), ())),
            preferred_element_type=jnp.float32,
        ) * SCALE
        m = jnp.max(s, axis=1, keepdims=True)
        p = jnp.exp(s - m)
        den = jnp.sum(p, axis=1, keepdims=True)
        c = jnp.dot(p, vr, preferred_element_type=jnp.float32) / den
        parts.append(c.reshape(8, 64, DH))
    return jnp.stack(parts, axis=1).reshape(SQ, DH)


def kernel(x, Wq, K_ext, V_ext, Wo):
    x2 = x.reshape(SQ, DM).astype(jnp.bfloat16)
    wq_bf = Wq.astype(jnp.bfloat16)
    wo_bf = Wo.astype(jnp.bfloat16)

    def body(x_ref, wq_ref, k_ref, v_ref, wo_ref, out_ref,
             gq, go, qbuf, cbuf, kbuf, vbuf,
             init_sems, kv_sems, send_sems, recv_sems):
        my = lax.axis_index("i")
        left = lax.rem(my + N_DEV - 1, N_DEV)
        right = lax.rem(my + 1, N_DEV)

        for half in range(2):
            pltpu.make_async_copy(
                wq_ref.at[:, pl.ds(half * HD, HD)], gq.at[my, half],
                init_sems.at[0, half],
            ).start()
            pltpu.make_async_copy(
                wo_ref.at[pl.ds(half * HD, HD), :], go.at[my, half],
                init_sems.at[1, half],
            ).start()
        out_ref[...] = jnp.zeros((SQ, DM), jnp.float32)
        for half in range(2):
            pltpu.make_async_copy(
                wq_ref.at[:, pl.ds(half * HD, HD)], gq.at[my, half],
                init_sems.at[0, half],
            ).wait()
            pltpu.make_async_copy(
                wo_ref.at[pl.ds(half * HD, HD), :], go.at[my, half],
                init_sems.at[1, half],
            ).wait()

        barrier_sem = pltpu.get_barrier_semaphore()
        for nbr in (left, right):
            pl.semaphore_signal(
                barrier_sem, inc=1,
                device_id=(nbr,), device_id_type=pl.DeviceIdType.MESH,
            )
        pl.semaphore_wait(barrier_sem, 2)

        def hop_rdmas(t):
            org_cw = lax.rem(my - t + N_DEV, N_DEV)
            org_ccw = lax.rem(my + t, N_DEV)
            rs = []
            for ti, ref in enumerate((gq, go)):
                rs.append(pltpu.make_async_remote_copy(
                    src_ref=ref.at[org_cw, 0], dst_ref=ref.at[org_cw, 0],
                    send_sem=send_sems.at[t, 0, ti],
                    recv_sem=recv_sems.at[t, 0, ti],
                    device_id=(right,), device_id_type=pl.DeviceIdType.MESH,
                ))
                rs.append(pltpu.make_async_remote_copy(
                    src_ref=ref.at[org_ccw, 1], dst_ref=ref.at[org_ccw, 1],
                    send_sem=send_sems.at[t, 1, ti],
                    recv_sem=recv_sems.at[t, 1, ti],
                    device_id=(left,), device_id_type=pl.DeviceIdType.MESH,
                ))
            return rs

        def compute_phase(p):
            if p == 0:
                grp_a = my
                grp_b = my
            else:
                grp_a = lax.rem(my - p + N_DEV, N_DEV)
                grp_b = lax.rem(my + p, N_DEV)
            q_a = jnp.dot(x_ref[...], gq[grp_a, 0],
                          preferred_element_type=jnp.float32)
            q_b = jnp.dot(x_ref[...], gq[grp_b, 1],
                          preferred_element_type=jnp.float32)
            for h in range(HH):
                qbuf[h] = q_a[:, h * DH:(h + 1) * DH]
                qbuf[HH + h] = q_b[:, h * DH:(h + 1) * DH]

            def gidx(h_i):
                return jnp.where(h_i < HH, grp_a, grp_b) * HG + h_i

            def kv_copies(h_i, slot):
                g = gidx(h_i)
                return (
                    pltpu.make_async_copy(
                        k_ref.at[my, :, g, :], kbuf.at[slot], kv_sems.at[slot, 0]
                    ),
                    pltpu.make_async_copy(
                        v_ref.at[my, :, g, :], vbuf.at[slot], kv_sems.at[slot, 1]
                    ),
                )

            for c in kv_copies(jnp.int32(0), 0):
                c.start()

            def head_body(h_i, carry):
                slot = lax.rem(h_i, 2)
                for c in kv_copies(h_i, slot):
                    c.wait()

                @pl.when(h_i < HG - 1)
                def _():
                    for c in kv_copies(h_i + 1, lax.rem(h_i + 1, 2)):
                        c.start()

                cbuf[h_i] = _attn_head(qbuf[h_i], kbuf[slot], vbuf[slot])
                return carry

            lax.fori_loop(0, HG, head_body, 0)

            ctx_a = jnp.concatenate([cbuf[h] for h in range(HH)], axis=1)
            ctx_b = jnp.concatenate([cbuf[HH + h] for h in range(HH)], axis=1)
            for lo in (0, SQ // 2):
                out_ref[pl.ds(lo, SQ // 2), :] += jnp.dot(
                    ctx_a[lo:lo + SQ // 2], go[grp_a, 0],
                    preferred_element_type=jnp.float32,
                ) + jnp.dot(
                    ctx_b[lo:lo + SQ // 2], go[grp_b, 1],
                    preferred_element_type=jnp.float32,
                )

        for t in range(NHOP):
            for r in hop_rdmas(t):
                r.start()
            compute_phase(t)
            for r in hop_rdmas(t):
                r.wait()
        compute_phase(NHOP)

    out = pl.pallas_call(
        body,
        out_shape=jax.ShapeDtypeStruct((SQ, DM), jnp.float32),
        in_specs=[
            pl.BlockSpec(memory_space=pltpu.MemorySpace.VMEM),
            pl.BlockSpec(memory_space=pl.ANY),
            pl.BlockSpec(memory_space=pl.ANY),
            pl.BlockSpec(memory_space=pl.ANY),
            pl.BlockSpec(memory_space=pl.ANY),
        ],
        out_specs=pl.BlockSpec(memory_space=pltpu.MemorySpace.VMEM),
        scratch_shapes=[
            pltpu.VMEM((N_DEV, 2, DM, HD), jnp.bfloat16),
            pltpu.VMEM((N_DEV, 2, HD, DM), jnp.bfloat16),
            pltpu.VMEM((HG, SQ, DH), jnp.float32),
            pltpu.VMEM((HG, SQ, DH), jnp.float32),
            pltpu.VMEM((2, SQ, DH), jnp.float32),
            pltpu.VMEM((2, SQ, DH), jnp.float32),
            pltpu.SemaphoreType.DMA((2, 2)),
            pltpu.SemaphoreType.DMA((2, 2)),
            pltpu.SemaphoreType.DMA((NHOP, 2, 2)),
            pltpu.SemaphoreType.DMA((NHOP, 2, 2)),
        ],
        compiler_params=pltpu.CompilerParams(
            collective_id=0,
            vmem_limit_bytes=100 * 1024 * 1024,
        ),
    )(x2, wq_bf, K_ext, V_ext, wo_bf)
    return out.reshape(1, SQ, DM)


# device time: 123262 ns/iter; 4.3993x vs baseline; 1.0360x over previous
import jax
import jax.numpy as jnp
from jax import lax
from jax.experimental import pallas as pl
from jax.experimental.pallas import tpu as pltpu

N_DEV = 4
SQ = 2048
DM = 1024
HQ = 32
HG = 8
HH = HG // 2
HD = HH * 128
DH = 128
SCALE = 0.08838834764831843
NHOP = N_DEV - 1


def _attn_head(q, k, v):
    q4 = q.reshape(8, 4, 64, DH)
    k4 = k.reshape(8, 4, 64, DH)
    v4 = v.reshape(8, 4, 64, DH)
    parts = []
    for r in range(4):
        qr = q4[:, r].reshape(512, DH)
        kr = k4[:, r].reshape(512, DH)
        vr = v4[:, r].reshape(512, DH)
        s = lax.dot_general(
            qr, kr, (((1,), (1,)), ((), ())),
            preferred_element_type=jnp.float32,
        ) * SCALE
        p = jnp.exp(s)
        den = jnp.sum(p, axis=1, keepdims=True)
        c = jnp.dot(p, vr, preferred_element_type=jnp.float32) / den
        parts.append(c.reshape(8, 64, DH))
    return jnp.stack(parts, axis=1).reshape(SQ, DH)


def kernel(x, Wq, K_ext, V_ext, Wo):
    x2 = x.reshape(SQ, DM).astype(jnp.bfloat16)
    wq_bf = Wq.astype(jnp.bfloat16)
    wo_bf = Wo.astype(jnp.bfloat16)

    def body(x_ref, wq_ref, k_ref, v_ref, wo_ref, out_ref,
             gq, go, qbuf, cbuf, kbuf, vbuf,
             init_sems, kv_sems, send_sems, recv_sems):
        my = lax.axis_index("i")
        left = lax.rem(my + N_DEV - 1, N_DEV)
        right = lax.rem(my + 1, N_DEV)

        def gidx_seq(s):
            p = lax.div(s, HG)
            h = lax.rem(s, HG)
            grp = jnp.where(
                h < HH,
                lax.rem(my - p + N_DEV, N_DEV),
                lax.rem(my + p, N_DEV),
            )
            return grp * HG + h

        def kv_copies(s):
            g = gidx_seq(s)
            slot = lax.rem(s, 4)
            return (
                pltpu.make_async_copy(
                    k_ref.at[my, :, g, :], kbuf.at[slot], kv_sems.at[slot, 0]
                ),
                pltpu.make_async_copy(
                    v_ref.at[my, :, g, :], vbuf.at[slot], kv_sems.at[slot, 1]
                ),
            )

        for s0 in range(3):
            for c in kv_copies(jnp.int32(s0)):
                c.start()

        for half in range(2):
            pltpu.make_async_copy(
                wq_ref.at[:, pl.ds(half * HD, HD)], gq.at[my, half],
                init_sems.at[0, half],
            ).start()
            pltpu.make_async_copy(
                wo_ref.at[pl.ds(half * HD, HD), :], go.at[my, half],
                init_sems.at[1, half],
            ).start()
        out_ref[...] = jnp.zeros((SQ, DM), jnp.float32)
        for half in range(2):
            pltpu.make_async_copy(
                wq_ref.at[:, pl.ds(half * HD, HD)], gq.at[my, half],
                init_sems.at[0, half],
            ).wait()
            pltpu.make_async_copy(
                wo_ref.at[pl.ds(half * HD, HD), :], go.at[my, half],
                init_sems.at[1, half],
            ).wait()

        barrier_sem = pltpu.get_barrier_semaphore()
        for nbr in (left, right):
            pl.semaphore_signal(
                barrier_sem, inc=1,
                device_id=(nbr,), device_id_type=pl.DeviceIdType.MESH,
            )
        pl.semaphore_wait(barrier_sem, 2)

        def hop_rdmas(t):
            org_cw = lax.rem(my - t + N_DEV, N_DEV)
            org_ccw = lax.rem(my + t, N_DEV)
            rs = []
            for ti, ref in enumerate((gq, go)):
                rs.append(pltpu.make_async_remote_copy(
                    src_ref=ref.at[org_cw, 0], dst_ref=ref.at[org_cw, 0],
                    send_sem=send_sems.at[t, 0, ti],
                    recv_sem=recv_sems.at[t, 0, ti],
                    device_id=(right,), device_id_type=pl.DeviceIdType.MESH,
                ))
                rs.append(pltpu.make_async_remote_copy(
                    src_ref=ref.at[org_ccw, 1], dst_ref=ref.at[org_ccw, 1],
                    send_sem=send_sems.at[t, 1, ti],
                    recv_sem=recv_sems.at[t, 1, ti],
                    device_id=(left,), device_id_type=pl.DeviceIdType.MESH,
                ))
            return rs

        def compute_phase(p):
            if p == 0:
                grp_a = my
                grp_b = my
            else:
                grp_a = lax.rem(my - p + N_DEV, N_DEV)
                grp_b = lax.rem(my + p, N_DEV)
            q_a = jnp.dot(x_ref[...], gq[grp_a, 0],
                          preferred_element_type=jnp.float32)
            q_b = jnp.dot(x_ref[...], gq[grp_b, 1],
                          preferred_element_type=jnp.float32)
            for h in range(HH):
                qbuf[h] = q_a[:, h * DH:(h + 1) * DH]
                qbuf[HH + h] = q_b[:, h * DH:(h + 1) * DH]

            def head_body(h_i, carry):
                s = p * HG + h_i
                slot = lax.rem(s, 4)
                for c in kv_copies(s):
                    c.wait()

                @pl.when(s + 3 < HQ)
                def _():
                    for c in kv_copies(s + 3):
                        c.start()

                cbuf[h_i] = _attn_head(qbuf[h_i], kbuf[slot], vbuf[slot])
                return carry

            lax.fori_loop(0, HG, head_body, 0)

            ctx_a = jnp.concatenate([cbuf[h] for h in range(HH)], axis=1)
            ctx_b = jnp.concatenate([cbuf[HH + h] for h in range(HH)], axis=1)
            for lo in (0, SQ // 2):
                out_ref[pl.ds(lo, SQ // 2), :] += jnp.dot(
                    ctx_a[lo:lo + SQ // 2], go[grp_a, 0],
                    preferred_element_type=jnp.float32,
                ) + jnp.dot(
                    ctx_b[lo:lo + SQ // 2], go[grp_b, 1],
                    preferred_element_type=jnp.float32,
                )

        for t in range(NHOP):
            for r in hop_rdmas(t):
                r.start()
            compute_phase(t)
            for r in hop_rdmas(t):
                r.wait()
        compute_phase(NHOP)

    out = pl.pallas_call(
        body,
        out_shape=jax.ShapeDtypeStruct((SQ, DM), jnp.float32),
        in_specs=[
            pl.BlockSpec(memory_space=pltpu.MemorySpace.VMEM),
            pl.BlockSpec(memory_space=pl.ANY),
            pl.BlockSpec(memory_space=pl.ANY),
            pl.BlockSpec(memory_space=pl.ANY),
            pl.BlockSpec(memory_space=pl.ANY),
        ],
        out_specs=pl.BlockSpec(memory_space=pltpu.MemorySpace.VMEM),
        scratch_shapes=[
            pltpu.VMEM((N_DEV, 2, DM, HD), jnp.bfloat16),
            pltpu.VMEM((N_DEV, 2, HD, DM), jnp.bfloat16),
            pltpu.VMEM((HG, SQ, DH), jnp.float32),
            pltpu.VMEM((HG, SQ, DH), jnp.float32),
            pltpu.VMEM((4, SQ, DH), jnp.float32),
            pltpu.VMEM((4, SQ, DH), jnp.float32),
            pltpu.SemaphoreType.DMA((2, 2)),
            pltpu.SemaphoreType.DMA((4, 2)),
            pltpu.SemaphoreType.DMA((NHOP, 2, 2)),
            pltpu.SemaphoreType.DMA((NHOP, 2, 2)),
        ],
        compiler_params=pltpu.CompilerParams(
            collective_id=0,
            vmem_limit_bytes=100 * 1024 * 1024,
        ),
    )(x2, wq_bf, K_ext, V_ext, wo_bf)
    return out.reshape(1, SQ, DM)


# device time: 116652 ns/iter; 4.6485x vs baseline; 1.0567x over previous
import jax
import jax.numpy as jnp
from jax import lax
from jax.experimental import pallas as pl
from jax.experimental.pallas import tpu as pltpu

N_DEV = 4
SQ = 2048
DM = 1024
HQ = 32
HG = 8
HH = HG // 2
HD = HH * 128
DH = 128
SCALE = 0.08838834764831843
NHOP = N_DEV - 1


def _attn_head(q, k, v):
    q4 = q.astype(jnp.float32).reshape(8, 4, 64, DH)
    k4 = k.reshape(8, 4, 64, DH)
    v4 = v.reshape(8, 4, 64, DH)
    parts = []
    for r in range(4):
        qr = q4[:, r].reshape(512, DH)
        kr = k4[:, r].reshape(512, DH)
        vr = v4[:, r].reshape(512, DH)
        s = lax.dot_general(
            qr, kr, (((1,), (1,)), ((), ())),
            preferred_element_type=jnp.float32,
        ) * SCALE
        p = jnp.exp(s)
        den = jnp.sum(p, axis=1, keepdims=True)
        c = jnp.dot(p, vr, preferred_element_type=jnp.float32) / den
        parts.append(c.reshape(8, 64, DH))
    return jnp.stack(parts, axis=1).reshape(SQ, DH)


def kernel(x, Wq, K_ext, V_ext, Wo):

    def body(x_ref, wq_ref, k_ref, v_ref, wo_ref, out_ref,
             gq, go, xbf, qbuf, cbuf, kbuf, vbuf, wqstage, wostage,
             init_sems, kv_sems, send_sems, recv_sems):
        my = lax.axis_index("i")
        left = lax.rem(my + N_DEV - 1, N_DEV)
        right = lax.rem(my + 1, N_DEV)

        def gidx_seq(s):
            p = lax.div(s, HG)
            h = lax.rem(s, HG)
            grp = jnp.where(
                h < HH,
                lax.rem(my - p + N_DEV, N_DEV),
                lax.rem(my + p, N_DEV),
            )
            return grp * HG + h

        def kv_copies(s):
            g = gidx_seq(s)
            slot = lax.rem(s, 4)
            return (
                pltpu.make_async_copy(
                    k_ref.at[my, :, g, :], kbuf.at[slot], kv_sems.at[slot, 0]
                ),
                pltpu.make_async_copy(
                    v_ref.at[my, :, g, :], vbuf.at[slot], kv_sems.at[slot, 1]
                ),
            )

        for s0 in range(3):
            for c in kv_copies(jnp.int32(s0)):
                c.start()

        def _stage(half):
            return (
                pltpu.make_async_copy(
                    wq_ref.at[:, pl.ds(half * HD, HD)], wqstage,
                    init_sems.at[0, half],
                ),
                pltpu.make_async_copy(
                    wo_ref.at[pl.ds(half * HD, HD), :], wostage,
                    init_sems.at[1, half],
                ),
            )

        for c in _stage(0):
            c.start()
        xbf[...] = x_ref[0].astype(jnp.bfloat16)
        out_ref[...] = jnp.zeros((1, SQ, DM), jnp.float32)
        for half in range(2):
            cq, co = _stage(half)
            cq.wait()
            gq[my, half] = wqstage[...].astype(jnp.bfloat16)
            co.wait()
            go[my, half] = wostage[...].astype(jnp.bfloat16)
            if half == 0:
                for c in _stage(1):
                    c.start()

        barrier_sem = pltpu.get_barrier_semaphore()
        for nbr in (left, right):
            pl.semaphore_signal(
                barrier_sem, inc=1,
                device_id=(nbr,), device_id_type=pl.DeviceIdType.MESH,
            )
        pl.semaphore_wait(barrier_sem, 2)

        def hop_rdmas(t):
            org_cw = lax.rem(my - t + N_DEV, N_DEV)
            org_ccw = lax.rem(my + t, N_DEV)
            rs = []
            for ti, ref in enumerate((gq, go)):
                rs.append(pltpu.make_async_remote_copy(
                    src_ref=ref.at[org_cw, 0], dst_ref=ref.at[org_cw, 0],
                    send_sem=send_sems.at[t, 0, ti],
                    recv_sem=recv_sems.at[t, 0, ti],
                    device_id=(right,), device_id_type=pl.DeviceIdType.MESH,
                ))
                rs.append(pltpu.make_async_remote_copy(
                    src_ref=ref.at[org_ccw, 1], dst_ref=ref.at[org_ccw, 1],
                    send_sem=send_sems.at[t, 1, ti],
                    recv_sem=recv_sems.at[t, 1, ti],
                    device_id=(left,), device_id_type=pl.DeviceIdType.MESH,
                ))
            return rs

        def compute_phase(p):
            if p == 0:
                grp_a = my
                grp_b = my
            else:
                grp_a = lax.rem(my - p + N_DEV, N_DEV)
                grp_b = lax.rem(my + p, N_DEV)
            q_a = jnp.dot(xbf[...], gq[grp_a, 0],
                          preferred_element_type=jnp.float32)
            for h in range(HH):
                qbuf[h] = q_a[:, h * DH:(h + 1) * DH].astype(jnp.bfloat16)
            q_b = jnp.dot(xbf[...], gq[grp_b, 1],
                          preferred_element_type=jnp.float32)
            for h in range(HH):
                qbuf[HH + h] = q_b[:, h * DH:(h + 1) * DH].astype(jnp.bfloat16)

            def head_body(h_i, carry):
                s = p * HG + h_i
                slot = lax.rem(s, 4)
                for c in kv_copies(s):
                    c.wait()

                @pl.when(s + 3 < HQ)
                def _():
                    for c in kv_copies(s + 3):
                        c.start()

                cbuf[h_i] = _attn_head(
                    qbuf[h_i], kbuf[slot], vbuf[slot]
                ).astype(jnp.bfloat16)
                return carry

            lax.fori_loop(0, HG, head_body, 0)

            ctx_a = jnp.concatenate([cbuf[h] for h in range(HH)], axis=1)
            ctx_b = jnp.concatenate([cbuf[HH + h] for h in range(HH)], axis=1)
            qr_rows = SQ // 4
            for qi in range(4):
                lo = qi * qr_rows
                out_ref[0, pl.ds(lo, qr_rows), :] += jnp.dot(
                    ctx_a[lo:lo + qr_rows], go[grp_a, 0],
                    preferred_element_type=jnp.float32,
                ) + jnp.dot(
                    ctx_b[lo:lo + qr_rows], go[grp_b, 1],
                    preferred_element_type=jnp.float32,
                )

        for t in range(NHOP):
            for r in hop_rdmas(t):
                r.start()
            compute_phase(t)
            for r in hop_rdmas(t):
                r.wait()
        compute_phase(NHOP)

    out = pl.pallas_call(
        body,
        out_shape=jax.ShapeDtypeStruct((1, SQ, DM), jnp.float32),
        in_specs=[
            pl.BlockSpec(memory_space=pltpu.MemorySpace.VMEM),
            pl.BlockSpec(memory_space=pl.ANY),
            pl.BlockSpec(memory_space=pl.ANY),
            pl.BlockSpec(memory_space=pl.ANY),
            pl.BlockSpec(memory_space=pl.ANY),
        ],
        out_specs=pl.BlockSpec(memory_space=pltpu.MemorySpace.VMEM),
        scratch_shapes=[
            pltpu.VMEM((N_DEV, 2, DM, HD), jnp.bfloat16),
            pltpu.VMEM((N_DEV, 2, HD, DM), jnp.bfloat16),
            pltpu.VMEM((SQ, DM), jnp.bfloat16),
            pltpu.VMEM((HG, SQ, DH), jnp.bfloat16),
            pltpu.VMEM((HG, SQ, DH), jnp.bfloat16),
            pltpu.VMEM((4, SQ, DH), jnp.float32),
            pltpu.VMEM((4, SQ, DH), jnp.float32),
            pltpu.VMEM((DM, HD), jnp.float32),
            pltpu.VMEM((HD, DM), jnp.float32),
            pltpu.SemaphoreType.DMA((2, 2)),
            pltpu.SemaphoreType.DMA((4, 2)),
            pltpu.SemaphoreType.DMA((NHOP, 2, 2)),
            pltpu.SemaphoreType.DMA((NHOP, 2, 2)),
        ],
        compiler_params=pltpu.CompilerParams(
            collective_id=0,
            vmem_limit_bytes=100 * 1024 * 1024,
        ),
    )(x, Wq, K_ext, V_ext, Wo)
    return out


# device time: 115274 ns/iter; 4.7041x vs baseline; 1.0120x over previous
import jax
import jax.numpy as jnp
from jax import lax
from jax.experimental import pallas as pl
from jax.experimental.pallas import tpu as pltpu

N_DEV = 4
SQ = 2048
DM = 1024
HQ = 32
HG = 8
HH = HG // 2
HD = HH * 128
DH = 128
SCALE = 0.08838834764831843
NHOP = N_DEV - 1


def _attn_head(q, k, v):
    q4 = q.astype(jnp.float32).reshape(8, 4, 64, DH)
    k4 = k.reshape(8, 4, 64, DH)
    v4 = v.reshape(8, 4, 64, DH)
    parts = []
    for r in range(4):
        qr = q4[:, r].reshape(512, DH)
        kr = k4[:, r].reshape(512, DH)
        vr = v4[:, r].reshape(512, DH)
        s = lax.dot_general(
            qr, kr, (((1,), (1,)), ((), ())),
            preferred_element_type=jnp.float32,
        ) * SCALE
        p = jnp.exp(s)
        den = jnp.sum(p, axis=1, keepdims=True)
        c = jnp.dot(p, vr, preferred_element_type=jnp.float32) / den
        parts.append(c.reshape(8, 64, DH))
    return jnp.stack(parts, axis=1).reshape(SQ, DH)


def kernel(x, Wq, K_ext, V_ext, Wo):

    def body(x_ref, wq_ref, k_ref, v_ref, wo_ref, out_ref,
             gq, go, xbf, qbuf, cbuf, kbuf, vbuf, wqstage, wostage,
             init_sems, kv_sems, send_sems, recv_sems):
        my = lax.axis_index("i")
        left = lax.rem(my + N_DEV - 1, N_DEV)
        right = lax.rem(my + 1, N_DEV)

        def gidx_seq(s):
            p = lax.div(s, HG)
            h = lax.rem(s, HG)
            grp = jnp.where(
                h < HH,
                lax.rem(my - p + N_DEV, N_DEV),
                lax.rem(my + p, N_DEV),
            )
            return grp * HG + h

        def kv_copies(s):
            g = gidx_seq(s)
            slot = lax.rem(s, 4)
            return (
                pltpu.make_async_copy(
                    k_ref.at[my, :, g, :], kbuf.at[slot], kv_sems.at[slot, 0]
                ),
                pltpu.make_async_copy(
                    v_ref.at[my, :, g, :], vbuf.at[slot], kv_sems.at[slot, 1]
                ),
            )

        for s0 in range(2):
            for c in kv_copies(jnp.int32(s0)):
                c.start()

        def _stage(half):
            return (
                pltpu.make_async_copy(
                    wq_ref.at[:, pl.ds(half * HD, HD)], wqstage,
                    init_sems.at[0, half],
                ),
                pltpu.make_async_copy(
                    wo_ref.at[pl.ds(half * HD, HD), :], wostage,
                    init_sems.at[1, half],
                ),
            )

        for c in _stage(0):
            c.start()
        xbf[...] = x_ref[0].astype(jnp.bfloat16)
        out_ref[...] = jnp.zeros((1, SQ, DM), jnp.float32)
        for half in range(2):
            cq, co = _stage(half)
            cq.wait()
            gq[my, half] = wqstage[...].astype(jnp.bfloat16)
            co.wait()
            go[my, half] = wostage[...].astype(jnp.bfloat16)
            if half == 0:
                for c in _stage(1):
                    c.start()

        barrier_sem = pltpu.get_barrier_semaphore()
        for nbr in (left, right):
            pl.semaphore_signal(
                barrier_sem, inc=1,
                device_id=(nbr,), device_id_type=pl.DeviceIdType.MESH,
            )
        pl.semaphore_wait(barrier_sem, 2)

        def hop_rdmas(t):
            org_cw = lax.rem(my - t + N_DEV, N_DEV)
            org_ccw = lax.rem(my + t, N_DEV)
            rs = []
            for ti, ref in enumerate((gq, go)):
                rs.append(pltpu.make_async_remote_copy(
                    src_ref=ref.at[org_cw, 0], dst_ref=ref.at[org_cw, 0],
                    send_sem=send_sems.at[t, 0, ti],
                    recv_sem=recv_sems.at[t, 0, ti],
                    device_id=(right,), device_id_type=pl.DeviceIdType.MESH,
                ))
                rs.append(pltpu.make_async_remote_copy(
                    src_ref=ref.at[org_ccw, 1], dst_ref=ref.at[org_ccw, 1],
                    send_sem=send_sems.at[t, 1, ti],
                    recv_sem=recv_sems.at[t, 1, ti],
                    device_id=(left,), device_id_type=pl.DeviceIdType.MESH,
                ))
            return rs

        def compute_phase(p):
            if p == 0:
                grp_a = my
                grp_b = my
            else:
                grp_a = lax.rem(my - p + N_DEV, N_DEV)
                grp_b = lax.rem(my + p, N_DEV)
            q_a = jnp.dot(xbf[...], gq[grp_a, 0],
                          preferred_element_type=jnp.float32)
            for h in range(HH):
                qbuf[h] = q_a[:, h * DH:(h + 1) * DH].astype(jnp.bfloat16)
            q_b = jnp.dot(xbf[...], gq[grp_b, 1],
                          preferred_element_type=jnp.float32)
            for h in range(HH):
                qbuf[HH + h] = q_b[:, h * DH:(h + 1) * DH].astype(jnp.bfloat16)

            def head_body(hh, carry):
                s0 = p * HG + 2 * hh
                for d in range(2):
                    for c in kv_copies(s0 + d):
                        c.wait()

                @pl.when(s0 + 2 < HQ)
                def _():
                    for d in range(2, 4):
                        for c in kv_copies(s0 + d):
                            c.start()

                for d in range(2):
                    h_i = 2 * hh + d
                    slot = lax.rem(s0 + d, 4)
                    cbuf[h_i] = _attn_head(
                        qbuf[h_i], kbuf[slot], vbuf[slot]
                    ).astype(jnp.bfloat16)
                return carry

            lax.fori_loop(0, HG // 2, head_body, 0)

            ctx_a = jnp.concatenate([cbuf[h] for h in range(HH)], axis=1)
            ctx_b = jnp.concatenate([cbuf[HH + h] for h in range(HH)], axis=1)
            qr_rows = SQ // 4
            for qi in range(4):
                lo = qi * qr_rows
                out_ref[0, pl.ds(lo, qr_rows), :] += jnp.dot(
                    ctx_a[lo:lo + qr_rows], go[grp_a, 0],
                    preferred_element_type=jnp.float32,
                ) + jnp.dot(
                    ctx_b[lo:lo + qr_rows], go[grp_b, 1],
                    preferred_element_type=jnp.float32,
                )

        for t in range(NHOP):
            for r in hop_rdmas(t):
                r.start()
            compute_phase(t)
            for r in hop_rdmas(t):
                r.wait()
        compute_phase(NHOP)

    out = pl.pallas_call(
        body,
        out_shape=jax.ShapeDtypeStruct((1, SQ, DM), jnp.float32),
        in_specs=[
            pl.BlockSpec(memory_space=pltpu.MemorySpace.VMEM),
            pl.BlockSpec(memory_space=pl.ANY),
            pl.BlockSpec(memory_space=pl.ANY),
            pl.BlockSpec(memory_space=pl.ANY),
            pl.BlockSpec(memory_space=pl.ANY),
        ],
        out_specs=pl.BlockSpec(memory_space=pltpu.MemorySpace.VMEM),
        scratch_shapes=[
            pltpu.VMEM((N_DEV, 2, DM, HD), jnp.bfloat16),
            pltpu.VMEM((N_DEV, 2, HD, DM), jnp.bfloat16),
            pltpu.VMEM((SQ, DM), jnp.bfloat16),
            pltpu.VMEM((HG, SQ, DH), jnp.bfloat16),
            pltpu.VMEM((HG, SQ, DH), jnp.bfloat16),
            pltpu.VMEM((4, SQ, DH), jnp.float32),
            pltpu.VMEM((4, SQ, DH), jnp.float32),
            pltpu.VMEM((DM, HD), jnp.float32),
            pltpu.VMEM((HD, DM), jnp.float32),
            pltpu.SemaphoreType.DMA((2, 2)),
            pltpu.SemaphoreType.DMA((4, 2)),
            pltpu.SemaphoreType.DMA((NHOP, 2, 2)),
            pltpu.SemaphoreType.DMA((NHOP, 2, 2)),
        ],
        compiler_params=pltpu.CompilerParams(
            collective_id=0,
            vmem_limit_bytes=100 * 1024 * 1024,
        ),
    )(x, Wq, K_ext, V_ext, Wo)
    return out
